# baseline (speedup 1.0000x reference)
import functools

import jax
import jax.numpy as jnp
from jax import lax
from jax.experimental import pallas as pl
from jax.experimental.pallas import tpu as pltpu
from jax.experimental.pallas import tpu_sc as plsc

D = 64
B = 16384
NF = 5
CH = 128
UV = 1000000
SV = 2 + 7 + 21 + 1000
SVP = (SV + 1) // 2

_info = plsc.get_sparse_core_info()
NC = _info.num_cores
NS = _info.num_subcores
NW = NC * NS
BPW = B // NW
NCH = BPW // CH
NR = NF * NCH
NRP = 24
T = NF * NCH

_mesh = plsc.VectorSubcoreMesh(core_axis_name="c", subcore_axis_name="s")


@functools.partial(
    pl.kernel,
    out_type=jax.ShapeDtypeStruct((NF, B, D), jnp.float32),
    mesh=_mesh,
    scratch_types=[
        pltpu.VMEM((2 * NRP, CH), jnp.int32),
        pltpu.VMEM_SHARED((SVP, 2 * D), jnp.float32),
        pltpu.VMEM((2, CH, 2 * D), jnp.float32),
        pltpu.VMEM((2, CH, D), jnp.float32),
        pltpu.SemaphoreType.DMA,
        pltpu.SemaphoreType.DMA,
        pltpu.SemaphoreType.DMA,
        pltpu.SemaphoreType.DMA,
    ],
)
def _emb_concat(idxw_hbm, Wu2, Ws2, out_hbm,
                idx_v, spm, pbuf, ebuf, sg0, sg1, sw0, sw1):
    gsems = (sg0, sg1)
    wsems = (sw0, sw1)

    sid = lax.axis_index("s")
    wid = sid * NC + lax.axis_index("c")
    base = wid * BPW

    @pl.when(sid == 0)
    def _():
        pltpu.sync_copy(Ws2, spm)

    pltpu.sync_copy(idxw_hbm.at[wid], idx_v)

    plsc.subcore_barrier()

    def gstart(t):
        f, c = divmod(t, NCH)
        src = Wu2 if f == 0 else spm
        return pltpu.async_copy(
            src.at[idx_v.at[NRP + f * NCH + c]], pbuf.at[t % 2],
            gsems[t % 2])

    def extract(t):
        f, c = divmod(t, NCH)
        pb = pbuf.at[t % 2]
        eb = ebuf.at[t % 2]
        row = f * NCH + c

        def gbody(g, _):
            u16 = idx_v[row, pl.ds(g * 16, 16)]
            h16 = (u16 & 1) * D
            for l in range(16):
                b = g * 16 + l
                h = h16[l]
                for q in range(D // 16):
                    eb[b, pl.ds(q * 16, 16)] = pb[b, pl.ds(h + q * 16, 16)]
            return _

        lax.fori_loop(0, CH // 16, gbody, 0)

    def wstart(t):
        f, c = divmod(t, NCH)
        return pltpu.async_copy(
            ebuf.at[t % 2],
            out_hbm.at[f, pl.ds(base + c * CH, CH)],
            wsems[t % 2])

    gcs = [None] * T
    wcs = [None] * T
    gcs[0] = gstart(0)
    for t in range(T):
        if t + 1 < T:
            if t - 1 >= 0:
                wcs[t - 1].wait()
            gcs[t + 1] = gstart(t + 1)
        gcs[t].wait()
        extract(t)
        wcs[t] = wstart(t)
    wcs[T - 2].wait()
    wcs[T - 1].wait()


def kernel(user_id, gender, age, occupation, zip_code,
           W_user_id, W_gender, W_age, W_occupation, W_zip_code):
    idx = jnp.stack([user_id, gender, age + 2, occupation + 9,
                     zip_code + 30])
    idx = idx.reshape(NF, NW, NCH, CH).transpose(1, 0, 2, 3)
    idx = idx.reshape(NW, NR, CH)
    pad = jnp.zeros((NW, NRP - NR, CH), jnp.int32)
    idxw = jnp.concatenate([idx, pad, idx >> 1, pad], axis=1)

    Ws = jnp.concatenate([W_gender, W_age, W_occupation, W_zip_code], axis=0)
    Ws2 = jnp.concatenate(
        [Ws, jnp.zeros((2 * SVP - SV, D), jnp.float32)]).reshape(SVP, 2 * D)

    out = _emb_concat(idxw, W_user_id.reshape(UV // 2, 2 * D), Ws2)
    return out.transpose(1, 0, 2).reshape(B, NF * D)
